# baseline (device time: 23096 ns/iter reference)
import jax
import jax.numpy as jnp
from jax import lax
from jax.experimental import pallas as pl
from jax.experimental.pallas import tpu as pltpu

N_DEV = 4
EPS = 1e-5
LANES = 128
CHUNK = 1024


def kernel(x, gamma):
    m, n_local = x.shape
    n_global = n_local * N_DEV
    kc = m // CHUNK
    csub = CHUNK // LANES

    gamma2 = gamma.reshape(1, n_local)

    def body(
        x_hbm, g_ref, out_hbm,
        xv, xg, comm_ref, outv,
        in_sems, out_sems, send_sems, recv_sems,
    ):
        my = lax.axis_index("i")
        gf = g_ref[:, :].astype(jnp.float32)

        in_copies = []
        for k in range(kc):
            rows = pl.ds(k * CHUNK, CHUNK)
            cp = pltpu.make_async_copy(x_hbm.at[rows, :], xv.at[rows, :],
                                       in_sems.at[k])
            cp.start()
            in_copies.append(cp)

        barrier_sem = pltpu.get_barrier_semaphore()
        for d in range(1, N_DEV):
            nbr = lax.rem(my + d, N_DEV)
            pl.semaphore_signal(
                barrier_sem, inc=1,
                device_id=(nbr,), device_id_type=pl.DeviceIdType.MESH,
            )
        pl.semaphore_wait(barrier_sem, N_DEV - 1)

        for k in range(kc):
            in_copies[k].wait()
            rows = pl.ds(k * CHUNK, CHUNK)
            x3 = xv[rows, :].reshape(csub, LANES, n_local)
            comm_ref[my, k] = jnp.sum(x3 * x3, axis=2)

        sends = []
        for d in range(1, N_DEV):
            dst = lax.rem(my + d, N_DEV)
            rdma = pltpu.make_async_remote_copy(
                src_ref=comm_ref.at[my],
                dst_ref=comm_ref.at[my],
                send_sem=send_sems.at[d - 1],
                recv_sem=recv_sems.at[my],
                device_id=(dst,),
                device_id_type=pl.DeviceIdType.MESH,
            )
            rdma.start()
            sends.append(rdma)

        xg[:, :] = xv[:, :] * gf

        for d in (1, 3, 2):
            src = lax.rem(my + d, N_DEV)
            recv = pltpu.make_async_remote_copy(
                src_ref=comm_ref.at[src],
                dst_ref=comm_ref.at[src],
                send_sem=send_sems.at[d - 1],
                recv_sem=recv_sems.at[src],
                device_id=(my,),
                device_id_type=pl.DeviceIdType.MESH,
            )
            recv.wait_recv()

        out_copies = [None, None]
        for k in range(kc):
            total = (comm_ref[0, k] + comm_ref[1, k]
                     + comm_ref[2, k] + comm_ref[3, k])
            inv = lax.rsqrt(total / n_global + EPS)
            big = lax.broadcast_in_dim(inv, (csub, LANES, n_local), (0, 1))
            rows = pl.ds(k * CHUNK, CHUNK)
            g3 = xg[rows, :].reshape(csub, LANES, n_local)
            slot = k % 2
            if out_copies[slot] is not None:
                out_copies[slot].wait()
            outv[slot] = (g3 * big).reshape(CHUNK, n_local).astype(jnp.bfloat16)
            cp = pltpu.make_async_copy(outv.at[slot], out_hbm.at[rows, :],
                                       out_sems.at[slot])
            cp.start()
            out_copies[slot] = cp
        for cp in out_copies:
            cp.wait()

        for rdma in sends:
            rdma.wait_send()

    return pl.pallas_call(
        body,
        out_shape=jax.ShapeDtypeStruct((m, n_local), jnp.bfloat16),
        in_specs=[
            pl.BlockSpec(memory_space=pl.ANY),
            pl.BlockSpec(memory_space=pltpu.VMEM),
        ],
        out_specs=pl.BlockSpec(memory_space=pl.ANY),
        scratch_shapes=[
            pltpu.VMEM((m, n_local), jnp.float32),
            pltpu.VMEM((m, n_local), jnp.float32),
            pltpu.VMEM((N_DEV, kc, csub, LANES), jnp.float32),
            pltpu.VMEM((2, CHUNK, n_local), jnp.bfloat16),
            pltpu.SemaphoreType.DMA((kc,)),
            pltpu.SemaphoreType.DMA((2,)),
            pltpu.SemaphoreType.DMA((N_DEV - 1,)),
            pltpu.SemaphoreType.DMA((N_DEV,)),
        ],
        compiler_params=pltpu.CompilerParams(
            vmem_limit_bytes=64 * 1024 * 1024,
            collective_id=0,
        ),
    )(x, gamma2)


# device time: 21111 ns/iter; 1.0940x vs baseline; 1.0940x over previous
import jax
import jax.numpy as jnp
from jax import lax
from jax.experimental import pallas as pl
from jax.experimental.pallas import tpu as pltpu

N_DEV = 4
EPS = 1e-5
LANES = 128
CHUNK = 512
LAG = 3


def kernel(x, gamma):
    m, n_local = x.shape
    n_global = n_local * N_DEV
    kc = m // CHUNK
    csub = CHUNK // LANES

    gamma2 = gamma.reshape(1, n_local)

    def body(
        x_hbm, g_ref, out_hbm,
        xv, comm_ref, outv,
        in_sems, out_sems, send_sems, recv_sems,
    ):
        my = lax.axis_index("i")
        gf = g_ref[:, :].astype(jnp.float32)

        in_copies = []
        for k in range(kc):
            rows = pl.ds(k * CHUNK, CHUNK)
            cp = pltpu.make_async_copy(x_hbm.at[rows, :], xv.at[rows, :],
                                       in_sems.at[k])
            cp.start()
            in_copies.append(cp)

        barrier_sem = pltpu.get_barrier_semaphore()
        for d in range(1, N_DEV):
            nbr = lax.rem(my + d, N_DEV)
            pl.semaphore_signal(
                barrier_sem, inc=1,
                device_id=(nbr,), device_id_type=pl.DeviceIdType.MESH,
            )

        sends = []
        out_copies = [None, None]

        def phase2(k):
            for d in (1, 3, 2):
                src = lax.rem(my + d, N_DEV)
                recv = pltpu.make_async_remote_copy(
                    src_ref=comm_ref.at[src, k],
                    dst_ref=comm_ref.at[src, k],
                    send_sem=send_sems.at[d - 1, k],
                    recv_sem=recv_sems.at[src, k],
                    device_id=(my,),
                    device_id_type=pl.DeviceIdType.MESH,
                )
                recv.wait_recv()
            total = (comm_ref[0, k] + comm_ref[1, k]
                     + comm_ref[2, k] + comm_ref[3, k])
            inv = lax.rsqrt(total / n_global + EPS)
            big = lax.broadcast_in_dim(inv, (csub, LANES, n_local), (0, 1))
            rows = pl.ds(k * CHUNK, CHUNK)
            x3 = xv[rows, :].reshape(csub, LANES, n_local)
            slot = k % 2
            if out_copies[slot] is not None:
                out_copies[slot].wait()
            outv[slot] = ((x3 * big).reshape(CHUNK, n_local)
                          * gf).astype(jnp.bfloat16)
            cp = pltpu.make_async_copy(outv.at[slot], out_hbm.at[rows, :],
                                       out_sems.at[slot])
            cp.start()
            out_copies[slot] = cp

        for k in range(kc):
            in_copies[k].wait()
            rows = pl.ds(k * CHUNK, CHUNK)
            x3 = xv[rows, :].reshape(csub, LANES, n_local)
            tile = jnp.sum(x3 * x3, axis=2)
            comm_ref[my, k] = tile
            if k == 0:
                pl.semaphore_wait(barrier_sem, N_DEV - 1)
            for d in range(1, N_DEV):
                dst = lax.rem(my + d, N_DEV)
                rdma = pltpu.make_async_remote_copy(
                    src_ref=comm_ref.at[my, k],
                    dst_ref=comm_ref.at[my, k],
                    send_sem=send_sems.at[d - 1, k],
                    recv_sem=recv_sems.at[my, k],
                    device_id=(dst,),
                    device_id_type=pl.DeviceIdType.MESH,
                )
                rdma.start()
                sends.append(rdma)
            if k >= LAG:
                phase2(k - LAG)
        for k in range(max(kc - LAG, 0), kc):
            phase2(k)

        for cp in out_copies:
            cp.wait()
        for rdma in sends:
            rdma.wait_send()

    return pl.pallas_call(
        body,
        out_shape=jax.ShapeDtypeStruct((m, n_local), jnp.bfloat16),
        in_specs=[
            pl.BlockSpec(memory_space=pl.ANY),
            pl.BlockSpec(memory_space=pltpu.VMEM),
        ],
        out_specs=pl.BlockSpec(memory_space=pl.ANY),
        scratch_shapes=[
            pltpu.VMEM((m, n_local), jnp.float32),
            pltpu.VMEM((N_DEV, kc, csub, LANES), jnp.float32),
            pltpu.VMEM((2, CHUNK, n_local), jnp.bfloat16),
            pltpu.SemaphoreType.DMA((kc,)),
            pltpu.SemaphoreType.DMA((2,)),
            pltpu.SemaphoreType.DMA((N_DEV - 1, kc)),
            pltpu.SemaphoreType.DMA((N_DEV, kc)),
        ],
        compiler_params=pltpu.CompilerParams(
            vmem_limit_bytes=64 * 1024 * 1024,
            collective_id=0,
        ),
    )(x, gamma2)


# device time: 20759 ns/iter; 1.1126x vs baseline; 1.0170x over previous
import jax
import jax.numpy as jnp
from jax import lax
from jax.experimental import pallas as pl
from jax.experimental.pallas import tpu as pltpu

N_DEV = 4
EPS = 1e-5
LANES = 128
CHUNK = 512
LAG = 8


def kernel(x, gamma):
    m, n_local = x.shape
    n_global = n_local * N_DEV
    kc = m // CHUNK
    csub = CHUNK // LANES

    gamma2 = gamma.reshape(1, n_local)

    def body(
        x_hbm, g_ref, out_hbm,
        xv, comm_ref, outv,
        in_sems, out_sems, send_sems, recv_sems,
    ):
        my = lax.axis_index("i")
        gf = g_ref[:, :].astype(jnp.float32)

        in_copies = []
        for k in range(kc):
            rows = pl.ds(k * CHUNK, CHUNK)
            cp = pltpu.make_async_copy(x_hbm.at[rows, :], xv.at[rows, :],
                                       in_sems.at[k])
            cp.start()
            in_copies.append(cp)

        barrier_sem = pltpu.get_barrier_semaphore()
        for d in range(1, N_DEV):
            nbr = lax.rem(my + d, N_DEV)
            pl.semaphore_signal(
                barrier_sem, inc=1,
                device_id=(nbr,), device_id_type=pl.DeviceIdType.MESH,
            )

        sends = []
        out_copies = [None, None]

        def phase2(k):
            for d in (1, 3, 2):
                src = lax.rem(my + d, N_DEV)
                recv = pltpu.make_async_remote_copy(
                    src_ref=comm_ref.at[src, k],
                    dst_ref=comm_ref.at[src, k],
                    send_sem=send_sems.at[d - 1, k],
                    recv_sem=recv_sems.at[src, k],
                    device_id=(my,),
                    device_id_type=pl.DeviceIdType.MESH,
                )
                recv.wait_recv()
            total = (comm_ref[0, k] + comm_ref[1, k]
                     + comm_ref[2, k] + comm_ref[3, k])
            inv = lax.rsqrt(total / n_global + EPS)
            big = lax.broadcast_in_dim(inv, (csub, LANES, n_local), (0, 1))
            rows = pl.ds(k * CHUNK, CHUNK)
            x3 = xv[rows, :].reshape(csub, LANES, n_local)
            slot = k % 2
            if out_copies[slot] is not None:
                out_copies[slot].wait()
            outv[slot] = ((x3 * big).reshape(CHUNK, n_local)
                          * gf).astype(jnp.bfloat16)
            cp = pltpu.make_async_copy(outv.at[slot], out_hbm.at[rows, :],
                                       out_sems.at[slot])
            cp.start()
            out_copies[slot] = cp

        for k in range(kc):
            in_copies[k].wait()
            rows = pl.ds(k * CHUNK, CHUNK)
            x3 = xv[rows, :].reshape(csub, LANES, n_local)
            tile = jnp.sum(x3 * x3, axis=2)
            comm_ref[my, k] = tile
            if k == 0:
                pl.semaphore_wait(barrier_sem, N_DEV - 1)
            for d in range(1, N_DEV):
                dst = lax.rem(my + d, N_DEV)
                rdma = pltpu.make_async_remote_copy(
                    src_ref=comm_ref.at[my, k],
                    dst_ref=comm_ref.at[my, k],
                    send_sem=send_sems.at[d - 1, k],
                    recv_sem=recv_sems.at[my, k],
                    device_id=(dst,),
                    device_id_type=pl.DeviceIdType.MESH,
                )
                rdma.start()
                sends.append(rdma)
            if k >= LAG:
                phase2(k - LAG)
        for k in range(max(kc - LAG, 0), kc):
            phase2(k)

        for cp in out_copies:
            cp.wait()
        for rdma in sends:
            rdma.wait_send()

    return pl.pallas_call(
        body,
        out_shape=jax.ShapeDtypeStruct((m, n_local), jnp.bfloat16),
        in_specs=[
            pl.BlockSpec(memory_space=pl.ANY),
            pl.BlockSpec(memory_space=pltpu.VMEM),
        ],
        out_specs=pl.BlockSpec(memory_space=pl.ANY),
        scratch_shapes=[
            pltpu.VMEM((m, n_local), jnp.float32),
            pltpu.VMEM((N_DEV, kc, csub, LANES), jnp.float32),
            pltpu.VMEM((2, CHUNK, n_local), jnp.bfloat16),
            pltpu.SemaphoreType.DMA((kc,)),
            pltpu.SemaphoreType.DMA((2,)),
            pltpu.SemaphoreType.DMA((N_DEV - 1, kc)),
            pltpu.SemaphoreType.DMA((N_DEV, kc)),
        ],
        compiler_params=pltpu.CompilerParams(
            vmem_limit_bytes=64 * 1024 * 1024,
            collective_id=0,
        ),
    )(x, gamma2)
